# Initial kernel scaffold; baseline (speedup 1.0000x reference)
import jax
import jax.numpy as jnp
from jax.experimental import pallas as pl


def kernel(w0, b0, w_rest, b_rest, x):
    raise NotImplementedError("write your pallas kernel here")


# speedup vs baseline: 5.8146x; 5.8146x over previous
import functools

import jax
import jax.numpy as jnp
from jax.experimental import pallas as pl
from jax.experimental.pallas import tpu as pltpu

_SLOPE = 0.2


def _mlp_kernel(x_ref, w0t_ref, b0_ref, wmid_ref, wlast_ref, o_ref, g_ref, *,
                n_mid, cout, slope):
    tb = o_ref.shape[0]
    x = x_ref[0]
    h = w0t_ref[...] * x + b0_ref[...]
    h = jnp.maximum(h, slope * h)
    g_ref[0:cout, :] = h
    g_ref[cout:, :] = jnp.ones((g_ref.shape[0] - cout, tb), jnp.float32)

    for l in range(n_mid):
        h = jnp.dot(wmid_ref[l], g_ref[...],
                    preferred_element_type=jnp.float32)
        h = jnp.maximum(h, slope * h)
        g_ref[0:cout, :] = h

    out = jax.lax.dot_general(g_ref[...], wlast_ref[...],
                              (((0,), (0,)), ((), ())),
                              preferred_element_type=jnp.float32)
    o_ref[...] = jnp.maximum(out, slope * out)


def kernel(w0, b0, w_rest, b_rest, x):
    cin, cout = w0.shape
    n_rest = w_rest.shape[0]
    n_mid = n_rest - 1
    B = x.shape[0]

    tb = 4096
    G = pl.cdiv(B, tb)
    Bp = G * tb

    xf = x.astype(jnp.float32).reshape(B)
    if Bp != B:
        xf = jnp.pad(xf, (0, Bp - B))
    xr = xf.reshape(G, 1, tb)

    kaug = cout + 8
    wmid = jnp.zeros((max(n_mid, 1), cout, kaug), jnp.float32)
    wmid = wmid.at[:, :, :cout].set(jnp.transpose(w_rest[:n_mid], (0, 2, 1)))
    wmid = wmid.at[:, :, cout].set(b_rest[:n_mid])
    wlast = jnp.zeros((kaug, cout), jnp.float32)
    wlast = wlast.at[:cout, :].set(w_rest[n_rest - 1])
    wlast = wlast.at[cout, :].set(b_rest[n_rest - 1])
    w0t = w0.astype(jnp.float32).reshape(cin, cout).T
    b0c = b0.astype(jnp.float32).reshape(cout, 1)

    out = pl.pallas_call(
        functools.partial(_mlp_kernel, n_mid=n_mid, cout=cout, slope=_SLOPE),
        out_shape=jax.ShapeDtypeStruct((Bp, cout), jnp.float32),
        grid=(G,),
        in_specs=[
            pl.BlockSpec((1, 1, tb), lambda i: (i, 0, 0)),
            pl.BlockSpec((cout, cin), lambda i: (0, 0)),
            pl.BlockSpec((cout, 1), lambda i: (0, 0)),
            pl.BlockSpec((max(n_mid, 1), cout, kaug), lambda i: (0, 0, 0)),
            pl.BlockSpec((kaug, cout), lambda i: (0, 0)),
        ],
        out_specs=pl.BlockSpec((tb, cout), lambda i: (i, 0)),
        scratch_shapes=[pltpu.VMEM((kaug, tb), jnp.float32)],
        compiler_params=pltpu.CompilerParams(
            dimension_semantics=("parallel",),
            vmem_limit_bytes=64 * 1024 * 1024,
        ),
    )(xr, w0t, b0c, wmid, wlast)
    return out[:B]


# speedup vs baseline: 6.2770x; 1.0795x over previous
import functools

import jax
import jax.numpy as jnp
from jax.experimental import pallas as pl
from jax.experimental.pallas import tpu as pltpu

_SLOPE = 0.2


def _mlp_kernel(x_ref, w0t_ref, b0_ref, wmid_ref, wlast_ref, o_ref, g_ref, *,
                n_mid, cout, slope):
    tb = o_ref.shape[0]
    x = x_ref[0]
    h = w0t_ref[...] * x + b0_ref[...]
    h = jnp.maximum(h, slope * h)
    g_ref[0:cout, :] = h
    g_ref[cout:, :] = jnp.ones((g_ref.shape[0] - cout, tb), jnp.float32)

    for l in range(n_mid):
        h = jnp.dot(wmid_ref[l], g_ref[...],
                    preferred_element_type=jnp.float32)
        h = jnp.maximum(h, slope * h)
        g_ref[0:cout, :] = h

    out = jax.lax.dot_general(g_ref[...], wlast_ref[...],
                              (((0,), (0,)), ((), ())),
                              preferred_element_type=jnp.float32)
    o_ref[...] = jnp.maximum(out, slope * out)


def kernel(w0, b0, w_rest, b_rest, x):
    cin, cout = w0.shape
    n_rest = w_rest.shape[0]
    n_mid = n_rest - 1
    B = x.shape[0]

    tb = 8192
    G = pl.cdiv(B, tb)
    Bp = G * tb

    xf = x.astype(jnp.float32).reshape(B)
    if Bp != B:
        xf = jnp.pad(xf, (0, Bp - B))
    xr = xf.reshape(G, 1, tb)

    kaug = cout + 8
    wmid = jnp.zeros((max(n_mid, 1), cout, kaug), jnp.float32)
    wmid = wmid.at[:, :, :cout].set(jnp.transpose(w_rest[:n_mid], (0, 2, 1)))
    wmid = wmid.at[:, :, cout].set(b_rest[:n_mid])
    wlast = jnp.zeros((kaug, cout), jnp.float32)
    wlast = wlast.at[:cout, :].set(w_rest[n_rest - 1])
    wlast = wlast.at[cout, :].set(b_rest[n_rest - 1])
    w0t = w0.astype(jnp.float32).reshape(cin, cout).T
    b0c = b0.astype(jnp.float32).reshape(cout, 1)

    out = pl.pallas_call(
        functools.partial(_mlp_kernel, n_mid=n_mid, cout=cout, slope=_SLOPE),
        out_shape=jax.ShapeDtypeStruct((Bp, cout), jnp.float32),
        grid=(G,),
        in_specs=[
            pl.BlockSpec((1, 1, tb), lambda i: (i, 0, 0)),
            pl.BlockSpec((cout, cin), lambda i: (0, 0)),
            pl.BlockSpec((cout, 1), lambda i: (0, 0)),
            pl.BlockSpec((max(n_mid, 1), cout, kaug), lambda i: (0, 0, 0)),
            pl.BlockSpec((kaug, cout), lambda i: (0, 0)),
        ],
        out_specs=pl.BlockSpec((tb, cout), lambda i: (i, 0)),
        scratch_shapes=[pltpu.VMEM((kaug, tb), jnp.float32)],
        compiler_params=pltpu.CompilerParams(
            dimension_semantics=("parallel",),
            vmem_limit_bytes=64 * 1024 * 1024,
        ),
    )(xr, w0t, b0c, wmid, wlast)
    return out[:B]


# speedup vs baseline: 6.4514x; 1.0278x over previous
import functools

import jax
import jax.numpy as jnp
from jax.experimental import pallas as pl
from jax.experimental.pallas import tpu as pltpu

_SLOPE = 0.2


def _mlp_kernel(x_ref, w0t_ref, b0_ref, wmid_ref, wlast_ref, o_ref, g_ref, *,
                n_mid, cout, slope):
    tb = o_ref.shape[0]
    x = x_ref[0]
    h = w0t_ref[...] * x + b0_ref[...]
    h = jnp.maximum(h, slope * h)
    g_ref[0:cout, :] = h
    g_ref[cout:, :] = jnp.ones((g_ref.shape[0] - cout, tb), jnp.float32)

    for l in range(n_mid):
        h = jnp.dot(wmid_ref[l], g_ref[...],
                    preferred_element_type=jnp.float32)
        h = jnp.maximum(h, slope * h)
        g_ref[0:cout, :] = h

    out = jax.lax.dot_general(g_ref[...], wlast_ref[...],
                              (((0,), (0,)), ((), ())),
                              preferred_element_type=jnp.float32)
    o_ref[...] = jnp.maximum(out, slope * out)


def kernel(w0, b0, w_rest, b_rest, x):
    cin, cout = w0.shape
    n_rest = w_rest.shape[0]
    n_mid = n_rest - 1
    B = x.shape[0]

    tb = 16384
    G = pl.cdiv(B, tb)
    Bp = G * tb

    xf = x.astype(jnp.float32).reshape(B)
    if Bp != B:
        xf = jnp.pad(xf, (0, Bp - B))
    xr = xf.reshape(G, 1, tb)

    kaug = cout + 8
    wmid = jnp.zeros((max(n_mid, 1), cout, kaug), jnp.float32)
    wmid = wmid.at[:, :, :cout].set(jnp.transpose(w_rest[:n_mid], (0, 2, 1)))
    wmid = wmid.at[:, :, cout].set(b_rest[:n_mid])
    wlast = jnp.zeros((kaug, cout), jnp.float32)
    wlast = wlast.at[:cout, :].set(w_rest[n_rest - 1])
    wlast = wlast.at[cout, :].set(b_rest[n_rest - 1])
    w0t = w0.astype(jnp.float32).reshape(cin, cout).T
    b0c = b0.astype(jnp.float32).reshape(cout, 1)

    out = pl.pallas_call(
        functools.partial(_mlp_kernel, n_mid=n_mid, cout=cout, slope=_SLOPE),
        out_shape=jax.ShapeDtypeStruct((Bp, cout), jnp.float32),
        grid=(G,),
        in_specs=[
            pl.BlockSpec((1, 1, tb), lambda i: (i, 0, 0)),
            pl.BlockSpec((cout, cin), lambda i: (0, 0)),
            pl.BlockSpec((cout, 1), lambda i: (0, 0)),
            pl.BlockSpec((max(n_mid, 1), cout, kaug), lambda i: (0, 0, 0)),
            pl.BlockSpec((kaug, cout), lambda i: (0, 0)),
        ],
        out_specs=pl.BlockSpec((tb, cout), lambda i: (i, 0)),
        scratch_shapes=[pltpu.VMEM((kaug, tb), jnp.float32)],
        compiler_params=pltpu.CompilerParams(
            dimension_semantics=("parallel",),
            vmem_limit_bytes=64 * 1024 * 1024,
        ),
    )(xr, w0t, b0c, wmid, wlast)
    return out[:B]
